# baseline (speedup 1.0000x reference)
import jax
import jax.numpy as jnp
from jax.experimental import pallas as pl
from jax.experimental.pallas import tpu as pltpu

_N = 5000
_NPAD = 5120
_BI = 1000
_THRESHOLD = 0.8


def _contain_kernel(rows_ref, cols_ref, areas_ref, out_ref):
    x1i = rows_ref[:, 0:1]
    y1i = rows_ref[:, 1:2]
    x2i = rows_ref[:, 2:3]
    y2i = rows_ref[:, 3:4]
    x1j = cols_ref[0:1, :]
    y1j = cols_ref[1:2, :]
    x2j = cols_ref[2:3, :]
    y2j = cols_ref[3:4, :]
    m = (x1j >= x1i) & (y1j >= y1i) & (x2j <= x2i) & (y2j <= y2i)
    mf = m.astype(jnp.float32)
    s = jax.lax.dot_general(
        mf,
        areas_ref[:, :],
        (((1,), (0,)), ((), ())),
        preferred_element_type=jnp.float32,
    )[:, 0:1]
    ai = (x2i - x1i) * (y2i - y1i)
    keep = (s - ai) <= _THRESHOLD * (ai + 1e-9)
    out_ref[:, :] = rows_ref[:, :] * keep.astype(jnp.float32)


@jax.jit
def kernel(boxes):
    cols = jnp.zeros((8, _NPAD), jnp.float32).at[:4, :_N].set(boxes.T)
    area = (boxes[:, 2] - boxes[:, 0]) * (boxes[:, 3] - boxes[:, 1])
    areas_col = jnp.zeros((_NPAD, 8), jnp.float32).at[:_N, 0].set(area)
    return pl.pallas_call(
        _contain_kernel,
        grid=(_N // _BI,),
        in_specs=[
            pl.BlockSpec((_BI, 4), lambda i: (i, 0)),
            pl.BlockSpec((8, _NPAD), lambda i: (0, 0)),
            pl.BlockSpec((_NPAD, 8), lambda i: (0, 0)),
        ],
        out_specs=pl.BlockSpec((_BI, 4), lambda i: (i, 0)),
        out_shape=jax.ShapeDtypeStruct((_N, 4), jnp.float32),
        compiler_params=pltpu.CompilerParams(
            dimension_semantics=("parallel",),
        ),
    )(boxes, cols, areas_col)


# speedup vs baseline: 1.1124x; 1.1124x over previous
import jax
import jax.numpy as jnp
from jax.experimental import pallas as pl
from jax.experimental.pallas import tpu as pltpu

_N = 5000
_NPAD = 5120
_BI = 1000
_THRESHOLD = 0.8


def _contain_kernel(rows_ref, full_ref, out_ref, cols_ref):
    @pl.when(pl.program_id(0) == 0)
    def _build_cols():
        cols_ref[:, :] = jnp.zeros((8, _NPAD), jnp.float32)
        cols_ref[0:4, 0:_N] = jnp.transpose(full_ref[:, :])

    x1i = rows_ref[:, 0:1]
    y1i = rows_ref[:, 1:2]
    x2i = rows_ref[:, 2:3]
    y2i = rows_ref[:, 3:4]
    x1j = cols_ref[0:1, :]
    y1j = cols_ref[1:2, :]
    x2j = cols_ref[2:3, :]
    y2j = cols_ref[3:4, :]
    aj = (x2j - x1j) * (y2j - y1j)
    m = (x1j >= x1i) & (y1j >= y1i) & (x2j <= x2i) & (y2j <= y2i)
    s = jnp.sum(
        jnp.where(m, jnp.broadcast_to(aj, (_BI, _NPAD)), 0.0),
        axis=1,
        keepdims=True,
    )
    ai = (x2i - x1i) * (y2i - y1i)
    keep = (s - ai) <= _THRESHOLD * (ai + 1e-9)
    out_ref[:, :] = rows_ref[:, :] * keep.astype(jnp.float32)


@jax.jit
def kernel(boxes):
    return pl.pallas_call(
        _contain_kernel,
        grid=(_N // _BI,),
        in_specs=[
            pl.BlockSpec((_BI, 4), lambda i: (i, 0)),
            pl.BlockSpec((_N, 4), lambda i: (0, 0)),
        ],
        out_specs=pl.BlockSpec((_BI, 4), lambda i: (i, 0)),
        out_shape=jax.ShapeDtypeStruct((_N, 4), jnp.float32),
        scratch_shapes=[pltpu.VMEM((8, _NPAD), jnp.float32)],
        compiler_params=pltpu.CompilerParams(
            dimension_semantics=("arbitrary",),
        ),
    )(boxes, boxes)


# speedup vs baseline: 1.4417x; 1.2961x over previous
import jax
import jax.numpy as jnp
from jax.experimental import pallas as pl
from jax.experimental.pallas import tpu as pltpu

_N = 5000
_NPAD = 5120
_BI = 1000
_THRESHOLD = 0.8


def _contain_kernel(rows_ref, full_ref, out_ref, cols_ref):
    @pl.when(pl.program_id(0) == 0)
    def _build_cols():
        cols_ref[:, :] = jnp.zeros((8, _NPAD), jnp.float32)
        cols_ref[0:4, 0:_N] = jnp.transpose(full_ref[:, :])

    x1i = rows_ref[:, 0:1]
    y1i = rows_ref[:, 1:2]
    x2i = rows_ref[:, 2:3]
    y2i = rows_ref[:, 3:4]
    x1j = cols_ref[0:1, :]
    y1j = cols_ref[1:2, :]
    x2j = cols_ref[2:3, :]
    y2j = cols_ref[3:4, :]
    aj = (x2j - x1j) * (y2j - y1j)
    ajb = jnp.broadcast_to(aj, (_BI, _NPAD))
    z = jnp.zeros((_BI, _NPAD), jnp.float32)
    contrib = jnp.where(
        (x1j >= x1i) & (y1j >= y1i),
        jnp.where((x2j <= x2i) & (y2j <= y2i), ajb, z),
        z,
    )
    s = jnp.sum(contrib, axis=1, keepdims=True)
    ai = (x2i - x1i) * (y2i - y1i)
    keep = (s - ai) <= _THRESHOLD * (ai + 1e-9)
    out_ref[:, :] = rows_ref[:, :] * keep.astype(jnp.float32)


@jax.jit
def kernel(boxes):
    return pl.pallas_call(
        _contain_kernel,
        grid=(_N // _BI,),
        in_specs=[
            pl.BlockSpec((_BI, 4), lambda i: (i, 0)),
            pl.BlockSpec((_N, 4), lambda i: (0, 0)),
        ],
        out_specs=pl.BlockSpec((_BI, 4), lambda i: (i, 0)),
        out_shape=jax.ShapeDtypeStruct((_N, 4), jnp.float32),
        scratch_shapes=[pltpu.VMEM((8, _NPAD), jnp.float32)],
        compiler_params=pltpu.CompilerParams(
            dimension_semantics=("arbitrary",),
        ),
    )(boxes, boxes)
